# baseline (speedup 1.0000x reference)
import functools

import jax
import jax.numpy as jnp
from jax import lax
from jax.experimental import pallas as pl
from jax.experimental.pallas import tpu as pltpu
from jax.experimental.pallas import tpu_sc as plsc

N_NODES = 10000
N_EDGES = 160000
IN_DIM = 128
HIDDEN = 64
N_LAYERS = 4

NPAD = 10112
CHUNK = 128
EPAD = 163840
NCHUNKS = EPAD // CHUNK
NWORKERS = 32
CPW = NCHUNKS // NWORKERS
RPT = NPAD // 16
PDIM = 5 * HIDDEN
PDIMP = PDIM + 2


def _dot(a, b):
    return jnp.dot(a, b, preferred_element_type=jnp.float32)


def _bn(x, g, b, eps=1e-5):
    mu = jnp.mean(x, axis=0, keepdims=True)
    var = jnp.var(x, axis=0, keepdims=True)
    return g * (x - mu) / jnp.sqrt(var + eps) + b


def _embed_body(h_ref, wemb_ref, bemb_ref, w1t_ref, w1b_ref, b1_ref,
                h0_ref, a_ref, b_ref):
    h0 = _dot(h_ref[...], wemb_ref[...]) + bemb_ref[...]
    h0_ref[...] = h0
    a_ref[...] = _dot(h0, w1t_ref[...]) + b1_ref[...]
    b_ref[...] = _dot(h0, w1b_ref[...])


def _layer_body(h_ref, n0_ref, n1_ref, eps_ref, w1_ref, b1_ref, g1_ref,
                bb1_ref, w2_ref, b2_ref, gm_ref, bm_ref, gh_ref, bh_ref,
                w1t_ref, w1b_ref, b1n_ref, hout_ref, a_ref, b_ref):
    h_in = h_ref[...]
    x = (1.0 + eps_ref[0, 0]) * h_in + (n0_ref[...] + n1_ref[...])
    x = _dot(x, w1_ref[...]) + b1_ref[...]
    x = _bn(x, g1_ref[...], bb1_ref[...])
    x = jnp.maximum(x, 0.0)
    x = _dot(x, w2_ref[...]) + b2_ref[...]
    x = _bn(x, gm_ref[...], bm_ref[...])
    x = jnp.maximum(x, 0.0)
    x = _bn(x, gh_ref[...], bh_ref[...])
    x = jnp.maximum(x, 0.0)
    h = h_in + x
    hout_ref[...] = h
    a_ref[...] = _dot(h, w1t_ref[...]) + b1n_ref[...]
    b_ref[...] = _dot(h, w1b_ref[...])


def _node_out3():
    return (jax.ShapeDtypeStruct((N_NODES, HIDDEN), jnp.float32),
            jax.ShapeDtypeStruct((N_NODES, HIDDEN), jnp.float32),
            jax.ShapeDtypeStruct((N_NODES, HIDDEN), jnp.float32))


_embed_call = pl.pallas_call(_embed_body, out_shape=_node_out3())
_layer_call = pl.pallas_call(_layer_body, out_shape=_node_out3())


_MESH = plsc.VectorSubcoreMesh(core_axis_name="c", subcore_axis_name="s")
_SC_PARAMS = pltpu.CompilerParams(use_tc_tiling_on_sc=False,
                                  needs_layout_passes=False)


@functools.partial(
    pl.kernel,
    out_type=jax.ShapeDtypeStruct((2, NPAD, HIDDEN), jnp.float32),
    mesh=_MESH,
    scratch_types=[
        pltpu.VMEM((CPW, CHUNK), jnp.int32),
        pltpu.VMEM((CPW, CHUNK), jnp.int32),
        pltpu.VMEM((CHUNK, HIDDEN), jnp.float32),
        pltpu.VMEM((CHUNK, HIDDEN), jnp.float32),
        pltpu.VMEM((RPT, HIDDEN), jnp.float32),
        pltpu.VMEM_SHARED((NPAD, HIDDEN), jnp.float32),
        pltpu.SemaphoreType.DMA,
        pltpu.SemaphoreType.DMA,
    ],
    compiler_params=_SC_PARAMS,
)
def _segsum(hpad, src2d, dst2d, zeros, out, sidx, didx, rows0, rows1, stage,
            acc, sem0, sem1):
    cid = lax.axis_index("c")
    sid = lax.axis_index("s")
    w = cid * 16 + sid
    pltpu.sync_copy(zeros, stage)
    pltpu.sync_copy(stage, acc.at[pl.ds(sid * RPT, RPT), :])
    plsc.subcore_barrier()
    pltpu.sync_copy(src2d.at[pl.ds(w * CPW, CPW), :], sidx)
    pltpu.sync_copy(dst2d.at[pl.ds(w * CPW, CPW), :], didx)

    pltpu.async_copy(hpad.at[sidx.at[0]], rows0, sem0)

    def body(j2, carry):
        j = j2 * 2
        pltpu.make_async_copy(hpad.at[sidx.at[j]], rows0, sem0).wait()
        pltpu.async_copy(hpad.at[sidx.at[j + 1]], rows1, sem1)
        pltpu.sync_copy(rows0, acc.at[didx.at[j]], add=True)
        pltpu.make_async_copy(hpad.at[sidx.at[j]], rows1, sem1).wait()

        @pl.when(j + 2 < CPW)
        def _():
            pltpu.async_copy(hpad.at[sidx.at[j + 2]], rows0, sem0)

        pltpu.sync_copy(rows1, acc.at[didx.at[j + 1]], add=True)
        return carry

    lax.fori_loop(0, CPW // 2, body, 0)
    plsc.subcore_barrier()
    pltpu.sync_copy(acc.at[pl.ds(sid * RPT, RPT), :], stage)
    pltpu.sync_copy(stage, out.at[cid].at[pl.ds(sid * RPT, RPT), :])


@functools.partial(
    pl.kernel,
    out_type=jax.ShapeDtypeStruct((NCHUNKS, 2, CHUNK), jnp.float32),
    mesh=_MESH,
    scratch_types=[
        pltpu.VMEM((CPW, CHUNK), jnp.int32),
        pltpu.VMEM((CPW, CHUNK), jnp.int32),
        pltpu.VMEM((CHUNK, PDIM), jnp.float32),
        pltpu.VMEM((CHUNK, PDIM), jnp.float32),
        pltpu.VMEM((2, CHUNK), jnp.float32),
        pltpu.VMEM((PDIM,), jnp.float32),
        pltpu.VMEM((PDIM,), jnp.float32),
        pltpu.VMEM((16,), jnp.float32),
        pltpu.VMEM((16,), jnp.float32),
        pltpu.SemaphoreType.DMA,
        pltpu.SemaphoreType.DMA,
        pltpu.SemaphoreType.DMA,
        pltpu.SemaphoreType.DMA,
    ],
    compiler_params=_SC_PARAMS,
)
def _edge_score(apad, bpad, src2d, dst2d, w0bc_h, w1bc_h, b0_h, b1_h, out,
                sidx, didx, crows0, crows1, outv, w0v, w1v, b0v, b1v,
                sa0, sa1, sb0, sb1):
    cid = lax.axis_index("c")
    sid = lax.axis_index("s")
    w = cid * 16 + sid
    pltpu.sync_copy(src2d.at[pl.ds(w * CPW, CPW), :], sidx)
    pltpu.sync_copy(dst2d.at[pl.ds(w * CPW, CPW), :], didx)
    pltpu.sync_copy(w0bc_h, w0v)
    pltpu.sync_copy(w1bc_h, w1v)
    pltpu.sync_copy(b0_h, b0v)
    pltpu.sync_copy(b1_h, b1v)
    lanes = lax.iota(jnp.int32, 16)
    ngroups = CHUNK // 16

    def compute(buf, j):
        w0blk = [w0v[pl.ds(k * 16, 16)] for k in range(PDIM // 16)]
        w1blk = [w1v[pl.ds(k * 16, 16)] for k in range(PDIM // 16)]
        for g in range(ngroups):
            def e_body(e2, oo):
                o0, o1 = oo
                for u in range(2):
                    e16 = e2 * 2 + u
                    e = g * 16 + e16
                    s0 = b0v[...]
                    s1 = b1v[...]
                    for k in range(PDIM // 16):
                        c = buf[e, pl.ds(k * 16, 16)]
                        gg = jnp.maximum(c, 0.0)
                        s0 = s0 + gg * w0blk[k]
                        s1 = s1 + gg * w1blk[k]
                    t0 = jnp.sum(s0)
                    t1 = jnp.sum(s1)
                    m = lanes == e16
                    o0 = jnp.where(m, t0, o0)
                    o1 = jnp.where(m, t1, o1)
                return o0, o1

            z = jnp.zeros((16,), jnp.float32)
            o0, o1 = lax.fori_loop(0, 8, e_body, (z, z))
            outv[0, pl.ds(g * 16, 16)] = o0
            outv[1, pl.ds(g * 16, 16)] = o1
        pltpu.sync_copy(outv, out.at[w * CPW + j])

    c0 = crows0
    c1 = crows1
    pltpu.async_copy(apad.at[sidx.at[0]], c0, sa0)

    def body(j2, carry):
        j = j2 * 2
        pltpu.make_async_copy(apad.at[sidx.at[j]], c0, sa0).wait()
        pltpu.async_copy(bpad.at[didx.at[j]], c0, sb0, add=True)
        pltpu.async_copy(apad.at[sidx.at[j + 1]], c1, sa1)
        pltpu.make_async_copy(bpad.at[didx.at[j]], c0, sb0).wait()
        compute(crows0, j)
        pltpu.make_async_copy(apad.at[sidx.at[j + 1]], c1, sa1).wait()
        pltpu.async_copy(bpad.at[didx.at[j + 1]], c1, sb1, add=True)

        @pl.when(j + 2 < CPW)
        def _():
            pltpu.async_copy(apad.at[sidx.at[j + 2]], c0, sa0)

        pltpu.make_async_copy(bpad.at[didx.at[j + 1]], c1, sb1).wait()
        compute(crows1, j + 1)
        return carry

    lax.fori_loop(0, CPW // 2, body, 0)


def kernel(h, edge_index, e, params):
    del e
    p = params
    pred = p['pred']
    src = edge_index[0].astype(jnp.int32)
    dst = edge_index[1].astype(jnp.int32)
    padfill = jnp.full((EPAD - N_EDGES,), N_NODES, jnp.int32)
    src2d = jnp.concatenate([src, padfill]).reshape(NCHUNKS, CHUNK)
    dst2d = jnp.concatenate([dst, padfill]).reshape(NCHUNKS, CHUNK)
    zeros = jnp.zeros((RPT, HIDDEN), jnp.float32)
    padrows = jnp.zeros((NPAD - N_NODES, HIDDEN), jnp.float32)

    def r1(v):
        return v.reshape(1, -1)

    w1t = [pp['W1'][:HIDDEN] for pp in pred]
    w1b = [pp['W1'][HIDDEN:] for pp in pred]
    b1p = [r1(pp['b1']) for pp in pred]

    h0, a0, b0 = _embed_call(h, p['W_emb'], r1(p['b_emb']),
                             w1t[0], w1b[0], b1p[0])
    a_parts, b_parts = [a0], [b0]
    hl = h0
    for l in range(N_LAYERS):
        gp = p['gin'][l]
        hpad = jnp.concatenate([hl, padrows], axis=0)
        part = _segsum(hpad, src2d, dst2d, zeros)
        hl, al, bl = _layer_call(
            hl, part[0, :N_NODES], part[1, :N_NODES],
            jnp.reshape(gp['eps'], (1, 1)),
            gp['W1'], r1(gp['b1']), r1(gp['bn1_g']), r1(gp['bn1_b']),
            gp['W2'], r1(gp['b2']), r1(gp['bn_mlp_g']), r1(gp['bn_mlp_b']),
            r1(gp['bn_h_g']), r1(gp['bn_h_b']),
            w1t[l + 1], w1b[l + 1], b1p[l + 1])
        a_parts.append(al)
        b_parts.append(bl)

    apad = jnp.concatenate(
        [jnp.concatenate(a_parts, axis=1),
         jnp.zeros((NPAD - N_NODES, PDIM), jnp.float32)], axis=0)
    bpad = jnp.concatenate(
        [jnp.concatenate(b_parts, axis=1),
         jnp.zeros((NPAD - N_NODES, PDIM), jnp.float32)], axis=0)

    w2s = jnp.concatenate([pp['W2'] for pp in pred], axis=0)
    b2s = sum(pp['b2'] for pp in pred)
    w0bc = w2s[:, 0]
    w1bc = w2s[:, 1]
    b0v = jnp.full((16,), 1.0 / 16.0, jnp.float32) * b2s[0]
    b1v = jnp.full((16,), 1.0 / 16.0, jnp.float32) * b2s[1]

    outc = _edge_score(apad, bpad, src2d, dst2d, w0bc, w1bc, b0v, b1v)
    score = outc.transpose(0, 2, 1).reshape(EPAD, 2)[:N_EDGES]
    return score


# speedup vs baseline: 1.0547x; 1.0547x over previous
import functools

import jax
import jax.numpy as jnp
from jax import lax
from jax.experimental import pallas as pl
from jax.experimental.pallas import tpu as pltpu
from jax.experimental.pallas import tpu_sc as plsc

N_NODES = 10000
N_EDGES = 160000
IN_DIM = 128
HIDDEN = 64
N_LAYERS = 4

NPAD = 10112
CHUNK = 128
EPAD = 163840
NCHUNKS = EPAD // CHUNK
NWORKERS = 32
CPW = NCHUNKS // NWORKERS
RPT = NPAD // 16
PDIM = 5 * HIDDEN
PDIMP = PDIM + 2


def _dot(a, b):
    return jnp.dot(a, b, preferred_element_type=jnp.float32)


def _bn(x, g, b, eps=1e-5):
    mu = jnp.mean(x, axis=0, keepdims=True)
    var = jnp.var(x, axis=0, keepdims=True)
    return g * (x - mu) / jnp.sqrt(var + eps) + b


def _embed_body(h_ref, wemb_ref, bemb_ref, w1t_ref, w1b_ref, b1_ref,
                h0_ref, a_ref, b_ref):
    h0 = _dot(h_ref[...], wemb_ref[...]) + bemb_ref[...]
    h0_ref[...] = h0
    a_ref[...] = _dot(h0, w1t_ref[...]) + b1_ref[...]
    b_ref[...] = _dot(h0, w1b_ref[...])


def _layer_body(h_ref, n0_ref, n1_ref, eps_ref, w1_ref, b1_ref, g1_ref,
                bb1_ref, w2_ref, b2_ref, gm_ref, bm_ref, gh_ref, bh_ref,
                w1t_ref, w1b_ref, b1n_ref, hout_ref, a_ref, b_ref):
    h_in = h_ref[...]
    x = (1.0 + eps_ref[0, 0]) * h_in + (n0_ref[...] + n1_ref[...])
    x = _dot(x, w1_ref[...]) + b1_ref[...]
    x = _bn(x, g1_ref[...], bb1_ref[...])
    x = jnp.maximum(x, 0.0)
    x = _dot(x, w2_ref[...]) + b2_ref[...]
    x = _bn(x, gm_ref[...], bm_ref[...])
    x = jnp.maximum(x, 0.0)
    x = _bn(x, gh_ref[...], bh_ref[...])
    x = jnp.maximum(x, 0.0)
    h = h_in + x
    hout_ref[...] = h
    a_ref[...] = _dot(h, w1t_ref[...]) + b1n_ref[...]
    b_ref[...] = _dot(h, w1b_ref[...])


def _node_out3():
    return (jax.ShapeDtypeStruct((N_NODES, HIDDEN), jnp.float32),
            jax.ShapeDtypeStruct((N_NODES, HIDDEN), jnp.float32),
            jax.ShapeDtypeStruct((N_NODES, HIDDEN), jnp.float32))


_embed_call = pl.pallas_call(_embed_body, out_shape=_node_out3())
_layer_call = pl.pallas_call(_layer_body, out_shape=_node_out3())


_MESH = plsc.VectorSubcoreMesh(core_axis_name="c", subcore_axis_name="s")
_SC_PARAMS = pltpu.CompilerParams(use_tc_tiling_on_sc=False,
                                  needs_layout_passes=False)


@functools.partial(
    pl.kernel,
    out_type=jax.ShapeDtypeStruct((2, NPAD, HIDDEN), jnp.float32),
    mesh=_MESH,
    scratch_types=[
        pltpu.VMEM((CPW, CHUNK), jnp.int32),
        pltpu.VMEM((CPW, CHUNK), jnp.int32),
        pltpu.VMEM((CHUNK, HIDDEN), jnp.float32),
        pltpu.VMEM((CHUNK, HIDDEN), jnp.float32),
        pltpu.VMEM((RPT, HIDDEN), jnp.float32),
        pltpu.VMEM_SHARED((NPAD, HIDDEN), jnp.float32),
        pltpu.SemaphoreType.DMA,
        pltpu.SemaphoreType.DMA,
    ],
    compiler_params=_SC_PARAMS,
)
def _segsum(hpad, src2d, dst2d, zeros, out, sidx, didx, rows0, rows1, stage,
            acc, sem0, sem1):
    cid = lax.axis_index("c")
    sid = lax.axis_index("s")
    w = cid * 16 + sid
    pltpu.sync_copy(zeros, stage)
    pltpu.sync_copy(stage, acc.at[pl.ds(sid * RPT, RPT), :])
    plsc.subcore_barrier()
    pltpu.sync_copy(src2d.at[pl.ds(w * CPW, CPW), :], sidx)
    pltpu.sync_copy(dst2d.at[pl.ds(w * CPW, CPW), :], didx)

    pltpu.async_copy(hpad.at[sidx.at[0]], rows0, sem0)

    def body(j2, carry):
        j = j2 * 2
        pltpu.make_async_copy(hpad.at[sidx.at[j]], rows0, sem0).wait()
        pltpu.async_copy(hpad.at[sidx.at[j + 1]], rows1, sem1)
        pltpu.sync_copy(rows0, acc.at[didx.at[j]], add=True)
        pltpu.make_async_copy(hpad.at[sidx.at[j]], rows1, sem1).wait()

        @pl.when(j + 2 < CPW)
        def _():
            pltpu.async_copy(hpad.at[sidx.at[j + 2]], rows0, sem0)

        pltpu.sync_copy(rows1, acc.at[didx.at[j + 1]], add=True)
        return carry

    lax.fori_loop(0, CPW // 2, body, 0)
    plsc.subcore_barrier()
    pltpu.sync_copy(acc.at[pl.ds(sid * RPT, RPT), :], stage)
    pltpu.sync_copy(stage, out.at[cid].at[pl.ds(sid * RPT, RPT), :])


@functools.partial(
    pl.kernel,
    out_type=jax.ShapeDtypeStruct((NCHUNKS, 2, CHUNK), jnp.float32),
    mesh=_MESH,
    scratch_types=[
        pltpu.VMEM((CPW, CHUNK), jnp.int32),
        pltpu.VMEM((CPW, CHUNK), jnp.int32),
        pltpu.VMEM((CHUNK, PDIM), jnp.float32),
        pltpu.VMEM((CHUNK, PDIM), jnp.float32),
        pltpu.VMEM((2, CHUNK), jnp.float32),
        pltpu.VMEM((PDIM,), jnp.float32),
        pltpu.VMEM((PDIM,), jnp.float32),
        pltpu.VMEM((16,), jnp.float32),
        pltpu.VMEM((16,), jnp.float32),
        pltpu.SemaphoreType.DMA,
        pltpu.SemaphoreType.DMA,
        pltpu.SemaphoreType.DMA,
        pltpu.SemaphoreType.DMA,
        pltpu.SemaphoreType.DMA,
        pltpu.SemaphoreType.DMA,
    ],
    compiler_params=_SC_PARAMS,
)
def _edge_score(apad, bpad, src2d, dst2d, w0bc_h, w1bc_h, b0_h, b1_h, out,
                sidx, didx, crows0, crows1, outv, w0v, w1v, b0v, b1v,
                sa0, sa1, sb0, sb0b, sb1, sb1b):
    cid = lax.axis_index("c")
    sid = lax.axis_index("s")
    w = cid * 16 + sid
    pltpu.sync_copy(src2d.at[pl.ds(w * CPW, CPW), :], sidx)
    pltpu.sync_copy(dst2d.at[pl.ds(w * CPW, CPW), :], didx)
    pltpu.sync_copy(w0bc_h, w0v)
    pltpu.sync_copy(w1bc_h, w1v)
    pltpu.sync_copy(b0_h, b0v)
    pltpu.sync_copy(b1_h, b1v)
    lanes = lax.iota(jnp.int32, 16)
    ngroups = CHUNK // 16

    def compute(buf, glo, gn):
        w0blk = [w0v[pl.ds(k * 16, 16)] for k in range(PDIM // 16)]
        w1blk = [w1v[pl.ds(k * 16, 16)] for k in range(PDIM // 16)]
        for g in range(glo, glo + gn):
            def e_body(e16, oo):
                o0, o1 = oo
                e = g * 16 + e16
                s0 = b0v[...]
                s1 = b1v[...]
                for k in range(PDIM // 16):
                    c = buf[e, pl.ds(k * 16, 16)]
                    gg = jnp.maximum(c, 0.0)
                    s0 = s0 + gg * w0blk[k]
                    s1 = s1 + gg * w1blk[k]
                t0 = jnp.sum(s0)
                t1 = jnp.sum(s1)
                m = lanes == e16
                return jnp.where(m, t0, o0), jnp.where(m, t1, o1)

            z = jnp.zeros((16,), jnp.float32)
            o0, o1 = lax.fori_loop(0, 16, e_body, (z, z))
            outv[0, pl.ds(g * 16, 16)] = o0
            outv[1, pl.ds(g * 16, 16)] = o1

    HC = CHUNK // 2
    HG = ngroups // 2

    def start_b(buf, j, s_lo, s_hi):
        pltpu.async_copy(bpad.at[didx.at[j, pl.ds(0, HC)]],
                         buf.at[pl.ds(0, HC), :], s_lo, add=True)
        pltpu.async_copy(bpad.at[didx.at[j, pl.ds(HC, HC)]],
                         buf.at[pl.ds(HC, HC), :], s_hi, add=True)

    def wait_b(buf, j, sem, lo):
        pltpu.make_async_copy(bpad.at[didx.at[j, pl.ds(lo, HC)]],
                              buf.at[pl.ds(lo, HC), :], sem).wait()

    def run_chunk(buf, j, s_lo, s_hi):
        wait_b(buf, j, s_lo, 0)
        compute(buf, 0, HG)
        wait_b(buf, j, s_hi, HC)
        compute(buf, HG, HG)
        pltpu.sync_copy(outv, out.at[w * CPW + j])

    pltpu.async_copy(apad.at[sidx.at[0]], crows0, sa0)

    def body(j2, carry):
        j = j2 * 2
        pltpu.make_async_copy(apad.at[sidx.at[j]], crows0, sa0).wait()
        start_b(crows0, j, sb0, sb0b)
        pltpu.async_copy(apad.at[sidx.at[j + 1]], crows1, sa1)
        run_chunk(crows0, j, sb0, sb0b)
        pltpu.make_async_copy(apad.at[sidx.at[j + 1]], crows1, sa1).wait()
        start_b(crows1, j + 1, sb1, sb1b)

        @pl.when(j + 2 < CPW)
        def _():
            pltpu.async_copy(apad.at[sidx.at[j + 2]], crows0, sa0)

        run_chunk(crows1, j + 1, sb1, sb1b)
        return carry

    lax.fori_loop(0, CPW // 2, body, 0)


def kernel(h, edge_index, e, params):
    del e
    p = params
    pred = p['pred']
    src = edge_index[0].astype(jnp.int32)
    dst = edge_index[1].astype(jnp.int32)
    padfill = jnp.full((EPAD - N_EDGES,), N_NODES, jnp.int32)
    src2d = jnp.concatenate([src, padfill]).reshape(NCHUNKS, CHUNK)
    dst2d = jnp.concatenate([dst, padfill]).reshape(NCHUNKS, CHUNK)
    zeros = jnp.zeros((RPT, HIDDEN), jnp.float32)
    padrows = jnp.zeros((NPAD - N_NODES, HIDDEN), jnp.float32)

    def r1(v):
        return v.reshape(1, -1)

    w1t = [pp['W1'][:HIDDEN] for pp in pred]
    w1b = [pp['W1'][HIDDEN:] for pp in pred]
    b1p = [r1(pp['b1']) for pp in pred]

    h0, a0, b0 = _embed_call(h, p['W_emb'], r1(p['b_emb']),
                             w1t[0], w1b[0], b1p[0])
    a_parts, b_parts = [a0], [b0]
    hl = h0
    for l in range(N_LAYERS):
        gp = p['gin'][l]
        hpad = jnp.concatenate([hl, padrows], axis=0)
        part = _segsum(hpad, src2d, dst2d, zeros)
        hl, al, bl = _layer_call(
            hl, part[0, :N_NODES], part[1, :N_NODES],
            jnp.reshape(gp['eps'], (1, 1)),
            gp['W1'], r1(gp['b1']), r1(gp['bn1_g']), r1(gp['bn1_b']),
            gp['W2'], r1(gp['b2']), r1(gp['bn_mlp_g']), r1(gp['bn_mlp_b']),
            r1(gp['bn_h_g']), r1(gp['bn_h_b']),
            w1t[l + 1], w1b[l + 1], b1p[l + 1])
        a_parts.append(al)
        b_parts.append(bl)

    apad = jnp.concatenate(
        [jnp.concatenate(a_parts, axis=1),
         jnp.zeros((NPAD - N_NODES, PDIM), jnp.float32)], axis=0)
    bpad = jnp.concatenate(
        [jnp.concatenate(b_parts, axis=1),
         jnp.zeros((NPAD - N_NODES, PDIM), jnp.float32)], axis=0)

    w2s = jnp.concatenate([pp['W2'] for pp in pred], axis=0)
    b2s = sum(pp['b2'] for pp in pred)
    w0bc = w2s[:, 0]
    w1bc = w2s[:, 1]
    b0v = jnp.full((16,), 1.0 / 16.0, jnp.float32) * b2s[0]
    b1v = jnp.full((16,), 1.0 / 16.0, jnp.float32) * b2s[1]

    outc = _edge_score(apad, bpad, src2d, dst2d, w0bc, w1bc, b0v, b1v)
    score = outc.transpose(0, 2, 1).reshape(EPAD, 2)[:N_EDGES]
    return score


# speedup vs baseline: 1.0594x; 1.0044x over previous
import functools

import jax
import jax.numpy as jnp
from jax import lax
from jax.experimental import pallas as pl
from jax.experimental.pallas import tpu as pltpu
from jax.experimental.pallas import tpu_sc as plsc

N_NODES = 10000
N_EDGES = 160000
IN_DIM = 128
HIDDEN = 64
N_LAYERS = 4

NPAD = 10112
CHUNK = 128
EPAD = 163840
NCHUNKS = EPAD // CHUNK
NWORKERS = 32
CPW = NCHUNKS // NWORKERS
RPT = NPAD // 16
PDIM = 5 * HIDDEN
PDIMP = PDIM + 2


def _dot(a, b):
    return jnp.dot(a, b, preferred_element_type=jnp.float32)


def _bn(x, g, b, eps=1e-5):
    mu = jnp.mean(x, axis=0, keepdims=True)
    var = jnp.var(x, axis=0, keepdims=True)
    return g * (x - mu) / jnp.sqrt(var + eps) + b


def _embed_body(h_ref, wemb_ref, bemb_ref, w1t_ref, w1b_ref, b1_ref,
                h0_ref, a_ref, b_ref):
    h0 = _dot(h_ref[...], wemb_ref[...]) + bemb_ref[...]
    h0_ref[...] = h0
    a_ref[...] = _dot(h0, w1t_ref[...]) + b1_ref[...]
    b_ref[...] = _dot(h0, w1b_ref[...])


def _layer_body(h_ref, n0_ref, n1_ref, eps_ref, w1_ref, b1_ref, g1_ref,
                bb1_ref, w2_ref, b2_ref, gm_ref, bm_ref, gh_ref, bh_ref,
                w1t_ref, w1b_ref, b1n_ref, hout_ref, a_ref, b_ref):
    h_in = h_ref[...]
    x = (1.0 + eps_ref[0, 0]) * h_in + (n0_ref[...] + n1_ref[...])
    x = _dot(x, w1_ref[...]) + b1_ref[...]
    x = _bn(x, g1_ref[...], bb1_ref[...])
    x = jnp.maximum(x, 0.0)
    x = _dot(x, w2_ref[...]) + b2_ref[...]
    x = _bn(x, gm_ref[...], bm_ref[...])
    x = jnp.maximum(x, 0.0)
    x = _bn(x, gh_ref[...], bh_ref[...])
    x = jnp.maximum(x, 0.0)
    h = h_in + x
    hout_ref[...] = h
    a_ref[...] = _dot(h, w1t_ref[...]) + b1n_ref[...]
    b_ref[...] = _dot(h, w1b_ref[...])


def _node_out3():
    return (jax.ShapeDtypeStruct((N_NODES, HIDDEN), jnp.float32),
            jax.ShapeDtypeStruct((N_NODES, HIDDEN), jnp.float32),
            jax.ShapeDtypeStruct((N_NODES, HIDDEN), jnp.float32))


_embed_call = pl.pallas_call(_embed_body, out_shape=_node_out3())
_layer_call = pl.pallas_call(_layer_body, out_shape=_node_out3())


_MESH = plsc.VectorSubcoreMesh(core_axis_name="c", subcore_axis_name="s")
_SC_PARAMS = pltpu.CompilerParams(use_tc_tiling_on_sc=False,
                                  needs_layout_passes=False)


@functools.partial(
    pl.kernel,
    out_type=jax.ShapeDtypeStruct((2, NPAD, HIDDEN), jnp.float32),
    mesh=_MESH,
    scratch_types=[
        pltpu.VMEM((CPW, CHUNK), jnp.int32),
        pltpu.VMEM((CPW, CHUNK), jnp.int32),
        pltpu.VMEM((CHUNK, HIDDEN), jnp.float32),
        pltpu.VMEM((CHUNK, HIDDEN), jnp.float32),
        pltpu.VMEM_SHARED((NPAD, HIDDEN), jnp.float32),
        pltpu.SemaphoreType.DMA,
        pltpu.SemaphoreType.DMA,
    ],
    compiler_params=_SC_PARAMS,
)
def _segsum(hpad, src2d, dst2d, zeros, out, sidx, didx, rows0, rows1,
            acc, sem0, sem1):
    cid = lax.axis_index("c")
    sid = lax.axis_index("s")
    w = cid * 16 + sid
    pltpu.sync_copy(zeros, acc.at[pl.ds(sid * RPT, RPT), :])
    plsc.subcore_barrier()
    pltpu.sync_copy(src2d.at[pl.ds(w * CPW, CPW), :], sidx)
    pltpu.sync_copy(dst2d.at[pl.ds(w * CPW, CPW), :], didx)

    pltpu.async_copy(hpad.at[sidx.at[0]], rows0, sem0)

    def body(j2, carry):
        j = j2 * 2
        pltpu.make_async_copy(hpad.at[sidx.at[j]], rows0, sem0).wait()
        pltpu.async_copy(hpad.at[sidx.at[j + 1]], rows1, sem1)
        pltpu.sync_copy(rows0, acc.at[didx.at[j]], add=True)
        pltpu.make_async_copy(hpad.at[sidx.at[j]], rows1, sem1).wait()

        @pl.when(j + 2 < CPW)
        def _():
            pltpu.async_copy(hpad.at[sidx.at[j + 2]], rows0, sem0)

        pltpu.sync_copy(rows1, acc.at[didx.at[j + 1]], add=True)
        return carry

    lax.fori_loop(0, CPW // 2, body, 0)
    plsc.subcore_barrier()
    pltpu.sync_copy(acc.at[pl.ds(sid * RPT, RPT), :],
                    out.at[cid].at[pl.ds(sid * RPT, RPT), :])


@functools.partial(
    pl.kernel,
    out_type=jax.ShapeDtypeStruct((NCHUNKS, 2, CHUNK), jnp.float32),
    mesh=_MESH,
    scratch_types=[
        pltpu.VMEM((CPW, CHUNK), jnp.int32),
        pltpu.VMEM((CPW, CHUNK), jnp.int32),
        pltpu.VMEM((CHUNK, PDIM), jnp.float32),
        pltpu.VMEM((CHUNK, PDIM), jnp.float32),
        pltpu.VMEM((2, CHUNK), jnp.float32),
        pltpu.VMEM((PDIM,), jnp.float32),
        pltpu.VMEM((PDIM,), jnp.float32),
        pltpu.VMEM((16,), jnp.float32),
        pltpu.VMEM((16,), jnp.float32),
        pltpu.SemaphoreType.DMA,
        pltpu.SemaphoreType.DMA,
        pltpu.SemaphoreType.DMA,
        pltpu.SemaphoreType.DMA,
        pltpu.SemaphoreType.DMA,
        pltpu.SemaphoreType.DMA,
    ],
    compiler_params=_SC_PARAMS,
)
def _edge_score(apad, bpad, src2d, dst2d, w0bc_h, w1bc_h, b0_h, b1_h, out,
                sidx, didx, crows0, crows1, outv, w0v, w1v, b0v, b1v,
                sa0, sa1, sb0, sb0b, sb1, sb1b):
    cid = lax.axis_index("c")
    sid = lax.axis_index("s")
    w = cid * 16 + sid
    pltpu.sync_copy(src2d.at[pl.ds(w * CPW, CPW), :], sidx)
    pltpu.sync_copy(dst2d.at[pl.ds(w * CPW, CPW), :], didx)
    pltpu.sync_copy(w0bc_h, w0v)
    pltpu.sync_copy(w1bc_h, w1v)
    pltpu.sync_copy(b0_h, b0v)
    pltpu.sync_copy(b1_h, b1v)
    lanes = lax.iota(jnp.int32, 16)
    ngroups = CHUNK // 16

    def compute(buf, glo, gn):
        w0blk = [w0v[pl.ds(k * 16, 16)] for k in range(PDIM // 16)]
        w1blk = [w1v[pl.ds(k * 16, 16)] for k in range(PDIM // 16)]
        for g in range(glo, glo + gn):
            def e_body(e16, oo):
                o0, o1 = oo
                e = g * 16 + e16
                s0 = b0v[...]
                s1 = b1v[...]
                for k in range(PDIM // 16):
                    c = buf[e, pl.ds(k * 16, 16)]
                    gg = jnp.maximum(c, 0.0)
                    s0 = s0 + gg * w0blk[k]
                    s1 = s1 + gg * w1blk[k]
                t0 = jnp.sum(s0)
                t1 = jnp.sum(s1)
                m = lanes == e16
                return jnp.where(m, t0, o0), jnp.where(m, t1, o1)

            z = jnp.zeros((16,), jnp.float32)
            o0, o1 = lax.fori_loop(0, 16, e_body, (z, z))
            outv[0, pl.ds(g * 16, 16)] = o0
            outv[1, pl.ds(g * 16, 16)] = o1

    HC = CHUNK // 2
    HG = ngroups // 2

    def start_b(buf, j, s_lo, s_hi):
        pltpu.async_copy(bpad.at[didx.at[j, pl.ds(0, HC)]],
                         buf.at[pl.ds(0, HC), :], s_lo, add=True)
        pltpu.async_copy(bpad.at[didx.at[j, pl.ds(HC, HC)]],
                         buf.at[pl.ds(HC, HC), :], s_hi, add=True)

    def wait_b(buf, j, sem, lo):
        pltpu.make_async_copy(bpad.at[didx.at[j, pl.ds(lo, HC)]],
                              buf.at[pl.ds(lo, HC), :], sem).wait()

    def run_chunk(buf, j, s_lo, s_hi):
        wait_b(buf, j, s_lo, 0)
        compute(buf, 0, HG)
        wait_b(buf, j, s_hi, HC)
        compute(buf, HG, HG)
        pltpu.sync_copy(outv, out.at[w * CPW + j])

    pltpu.async_copy(apad.at[sidx.at[0]], crows0, sa0)

    def body(j2, carry):
        j = j2 * 2
        pltpu.make_async_copy(apad.at[sidx.at[j]], crows0, sa0).wait()
        start_b(crows0, j, sb0, sb0b)
        pltpu.async_copy(apad.at[sidx.at[j + 1]], crows1, sa1)
        run_chunk(crows0, j, sb0, sb0b)
        pltpu.make_async_copy(apad.at[sidx.at[j + 1]], crows1, sa1).wait()
        start_b(crows1, j + 1, sb1, sb1b)

        @pl.when(j + 2 < CPW)
        def _():
            pltpu.async_copy(apad.at[sidx.at[j + 2]], crows0, sa0)

        run_chunk(crows1, j + 1, sb1, sb1b)
        return carry

    lax.fori_loop(0, CPW // 2, body, 0)


def kernel(h, edge_index, e, params):
    del e
    p = params
    pred = p['pred']
    src = edge_index[0].astype(jnp.int32)
    dst = edge_index[1].astype(jnp.int32)
    padfill = jnp.full((EPAD - N_EDGES,), N_NODES, jnp.int32)
    src2d = jnp.concatenate([src, padfill]).reshape(NCHUNKS, CHUNK)
    dst2d = jnp.concatenate([dst, padfill]).reshape(NCHUNKS, CHUNK)
    zeros = jnp.zeros((RPT, HIDDEN), jnp.float32)
    padrows = jnp.zeros((NPAD - N_NODES, HIDDEN), jnp.float32)

    def r1(v):
        return v.reshape(1, -1)

    w1t = [pp['W1'][:HIDDEN] for pp in pred]
    w1b = [pp['W1'][HIDDEN:] for pp in pred]
    b1p = [r1(pp['b1']) for pp in pred]

    h0, a0, b0 = _embed_call(h, p['W_emb'], r1(p['b_emb']),
                             w1t[0], w1b[0], b1p[0])
    a_parts, b_parts = [a0], [b0]
    hl = h0
    for l in range(N_LAYERS):
        gp = p['gin'][l]
        hpad = jnp.concatenate([hl, padrows], axis=0)
        part = _segsum(hpad, src2d, dst2d, zeros)
        hl, al, bl = _layer_call(
            hl, part[0, :N_NODES], part[1, :N_NODES],
            jnp.reshape(gp['eps'], (1, 1)),
            gp['W1'], r1(gp['b1']), r1(gp['bn1_g']), r1(gp['bn1_b']),
            gp['W2'], r1(gp['b2']), r1(gp['bn_mlp_g']), r1(gp['bn_mlp_b']),
            r1(gp['bn_h_g']), r1(gp['bn_h_b']),
            w1t[l + 1], w1b[l + 1], b1p[l + 1])
        a_parts.append(al)
        b_parts.append(bl)

    apad = jnp.concatenate(
        [jnp.concatenate(a_parts, axis=1),
         jnp.zeros((NPAD - N_NODES, PDIM), jnp.float32)], axis=0)
    bpad = jnp.concatenate(
        [jnp.concatenate(b_parts, axis=1),
         jnp.zeros((NPAD - N_NODES, PDIM), jnp.float32)], axis=0)

    w2s = jnp.concatenate([pp['W2'] for pp in pred], axis=0)
    b2s = sum(pp['b2'] for pp in pred)
    w0bc = w2s[:, 0]
    w1bc = w2s[:, 1]
    b0v = jnp.full((16,), 1.0 / 16.0, jnp.float32) * b2s[0]
    b1v = jnp.full((16,), 1.0 / 16.0, jnp.float32) * b2s[1]

    outc = _edge_score(apad, bpad, src2d, dst2d, w0bc, w1bc, b0v, b1v)
    score = outc.transpose(0, 2, 1).reshape(EPAD, 2)[:N_EDGES]
    return score
